# baseline (speedup 1.0000x reference)
import functools

import jax
import jax.numpy as jnp
from jax import lax
from jax.experimental import pallas as pl
from jax.experimental.pallas import tpu as pltpu
from jax.experimental.pallas import tpu_sc as plsc

BATCH = 16384
VOCAB = 1000000
DIM = 16
NFIELDS = 4
PACK = 128 // DIM

_info = plsc.get_sparse_core_info()
_NC = _info.num_cores
_NS = _info.num_subcores
_NW = _NC * _NS
_BPW = BATCH // _NW
_CHUNK = 256

_mesh = plsc.VectorSubcoreMesh(core_axis_name="c", subcore_axis_name="s")


@functools.partial(
    pl.kernel,
    mesh=_mesh,
    out_type=tuple(
        jax.ShapeDtypeStruct((BATCH, DIM), jnp.float32) for _ in range(NFIELDS)
    ),
    scratch_types=[
        pltpu.VMEM((_CHUNK,), jnp.int32),
        pltpu.VMEM((_CHUNK,), jnp.int32),
        pltpu.VMEM((_CHUNK, 128), jnp.float32),
        pltpu.VMEM((_CHUNK, DIM), jnp.float32),
        pltpu.SemaphoreType.DMA,
    ],
)
def _gather4(t0, t1, t2, t3, i0, i1, i2, i3, o0, o1, o2, o3,
             idx_v, prow_v, packed_v, out_v, sem):
    wid = lax.axis_index("s") * _NC + lax.axis_index("c")
    base = wid * _BPW

    for t, i, o in ((t0, i0, o0), (t1, i1, o1), (t2, i2, o2), (t3, i3, o3)):
        for c in range(_BPW // _CHUNK):
            cbase = base + c * _CHUNK
            pltpu.sync_copy(i.at[pl.ds(cbase, _CHUNK)], idx_v)

            def _prow_body(g, _):
                v = idx_v[pl.ds(g * 16, 16)]
                prow_v[pl.ds(g * 16, 16)] = jax.lax.shift_right_logical(v, 3)
                return 0

            lax.fori_loop(0, _CHUNK // 16, _prow_body, 0, unroll=4)

            pltpu.async_copy(t.at[prow_v], packed_v, sem).wait()

            def _extract_body(g, _):
                kv = (idx_v[pl.ds(g * 16, 16)] & 7) * DIM
                b0 = g * 16
                for j in range(16):
                    out_v[b0 + j] = packed_v[b0 + j, pl.ds(kv[j], DIM)]
                return 0

            lax.fori_loop(0, _CHUNK // 16, _extract_body, 0)

            pltpu.sync_copy(out_v, o.at[pl.ds(cbase, _CHUNK)])


def _pack_tc(t, scale):
    tt = t.T
    t3 = tt.reshape(DIM, VOCAB // PACK, PACK)
    t4 = jnp.transpose(t3, (1, 2, 0)) * scale
    return t4.reshape(VOCAB // PACK, 128)


def kernel(feat0, feat1, feat2, feat3,
           table_feat0, table_feat1, table_feat2, table_feat3):
    scale = (feat0[0] * 0 + 1).astype(jnp.float32)
    return _gather4(
        _pack_tc(table_feat0, scale), _pack_tc(table_feat1, scale),
        _pack_tc(table_feat2, scale), _pack_tc(table_feat3, scale),
        feat0.astype(jnp.int32), feat1.astype(jnp.int32),
        feat2.astype(jnp.int32), feat3.astype(jnp.int32),
    )


# speedup vs baseline: 2.8811x; 2.8811x over previous
import functools

import jax
import jax.numpy as jnp
from jax import lax
from jax.experimental import pallas as pl
from jax.experimental.pallas import tpu as pltpu
from jax.experimental.pallas import tpu_sc as plsc

BATCH = 16384
VOCAB = 1000000
DIM = 16
NFIELDS = 4

_info = plsc.get_sparse_core_info()
_NC = _info.num_cores
_NS = _info.num_subcores
_NW = _NC * _NS
_BPW = BATCH // _NW
_CHUNK = 256

_mesh = plsc.VectorSubcoreMesh(core_axis_name="c", subcore_axis_name="s")


@functools.partial(
    pl.kernel,
    mesh=_mesh,
    out_type=tuple(
        jax.ShapeDtypeStruct((BATCH, DIM), jnp.float32) for _ in range(NFIELDS)
    ),
    scratch_types=[
        pltpu.VMEM((_CHUNK,), jnp.int32),
        pltpu.VMEM((16 * DIM, 128), jnp.float32),
        pltpu.VMEM((_CHUNK, DIM), jnp.float32),
        pltpu.SemaphoreType.DMA,
    ],
    compiler_params=pltpu.CompilerParams(needs_layout_passes=False),
)
def _gather4(t0, t1, t2, t3, i0, i1, i2, i3, o0, o1, o2, o3,
             idx_v, win_v, out_v, sem):
    wid = lax.axis_index("s") * _NC + lax.axis_index("c")
    base = wid * _BPW
    lane = lax.iota(jnp.int32, 16)

    for t, i, o in ((t0, i0, o0), (t1, i1, o1), (t2, i2, o2), (t3, i3, o3)):
        for c in range(_BPW // _CHUNK):
            cbase = base + c * _CHUNK
            pltpu.sync_copy(i.at[pl.ds(cbase, _CHUNK)], idx_v)

            def _group_body(g, _):
                kv = idx_v[pl.ds(g * 16, 16)]
                tc128 = jax.lax.shift_right_logical(kv, 7) * 128
                for j in range(16):
                    start = pl.multiple_of(tc128[j], 128)
                    pltpu.async_copy(
                        t.at[:, pl.ds(start, 128)],
                        win_v.at[pl.ds(j * DIM, DIM), :],
                        sem,
                    )
                for j in range(16):
                    pltpu.make_async_copy(
                        t.at[:, pl.ds(0, 128)],
                        win_v.at[pl.ds(j * DIM, DIM), :],
                        sem,
                    ).wait()
                lv = kv & 127
                for j in range(16):
                    vals = plsc.load_gather(
                        win_v, [j * DIM + lane, jnp.full((16,), lv[j], jnp.int32)]
                    )
                    out_v[g * 16 + j] = vals
                return 0

            lax.fori_loop(0, _CHUNK // 16, _group_body, 0)

            pltpu.sync_copy(out_v, o.at[pl.ds(cbase, _CHUNK)])


def kernel(feat0, feat1, feat2, feat3,
           table_feat0, table_feat1, table_feat2, table_feat3):
    return _gather4(
        table_feat0.T, table_feat1.T, table_feat2.T, table_feat3.T,
        feat0.astype(jnp.int32), feat1.astype(jnp.int32),
        feat2.astype(jnp.int32), feat3.astype(jnp.int32),
    )


# speedup vs baseline: 3.4422x; 1.1948x over previous
import functools

import jax
import jax.numpy as jnp
from jax import lax
from jax.experimental import pallas as pl
from jax.experimental.pallas import tpu as pltpu
from jax.experimental.pallas import tpu_sc as plsc

BATCH = 16384
VOCAB = 1000000
DIM = 16
NFIELDS = 4

_info = plsc.get_sparse_core_info()
_NC = _info.num_cores
_NS = _info.num_subcores
_NW = _NC * _NS
_BPW = BATCH // _NW
_CHUNK = 256
_NG = _CHUNK // 16

_mesh = plsc.VectorSubcoreMesh(core_axis_name="c", subcore_axis_name="s")


@functools.partial(
    pl.kernel,
    mesh=_mesh,
    out_type=tuple(
        jax.ShapeDtypeStruct((BATCH, DIM), jnp.float32) for _ in range(NFIELDS)
    ),
    scratch_types=[
        pltpu.VMEM((_CHUNK,), jnp.int32),
        pltpu.VMEM((2 * 16 * DIM, 128), jnp.float32),
        pltpu.VMEM((_CHUNK, DIM), jnp.float32),
        pltpu.SemaphoreType.DMA,
        pltpu.SemaphoreType.DMA,
    ],
    compiler_params=pltpu.CompilerParams(needs_layout_passes=False),
)
def _gather4(t0, t1, t2, t3, i0, i1, i2, i3, o0, o1, o2, o3,
             idx_v, win_v, out_v, sem0, sem1):
    wid = lax.axis_index("s") * _NC + lax.axis_index("c")
    base = wid * _BPW
    lane = lax.iota(jnp.int32, 16)

    for t, i, o in ((t0, i0, o0), (t1, i1, o1), (t2, i2, o2), (t3, i3, o3)):

        def _fire(g, par, sem):
            kv = idx_v[pl.ds(g * 16, 16)]
            tc128 = jax.lax.shift_right_logical(kv, 7) * 128
            for j in range(16):
                start = pl.multiple_of(tc128[j], 128)
                pltpu.async_copy(
                    t.at[:, pl.ds(start, 128)],
                    win_v.at[pl.ds((par * 16 + j) * DIM, DIM), :],
                    sem,
                )

        def _drain_extract(g, par, sem):
            for j in range(16):
                pltpu.make_async_copy(
                    t.at[:, pl.ds(0, 128)],
                    win_v.at[pl.ds((par * 16 + j) * DIM, DIM), :],
                    sem,
                ).wait()
            kv = idx_v[pl.ds(g * 16, 16)]
            lv = kv & 127
            for j in range(16):
                vals = plsc.load_gather(
                    win_v,
                    [(par * 16 + j) * DIM + lane,
                     jnp.full((16,), lv[j], jnp.int32)],
                )
                out_v[g * 16 + j] = vals

        for c in range(_BPW // _CHUNK):
            cbase = base + c * _CHUNK
            pltpu.sync_copy(i.at[pl.ds(cbase, _CHUNK)], idx_v)

            _fire(0, 0, sem0)

            def _pair_body(h, _):
                g0 = h * 2
                _fire(g0 + 1, 1, sem1)
                _drain_extract(g0, 0, sem0)

                @pl.when(h + 1 < _NG // 2)
                def _():
                    _fire(g0 + 2, 0, sem0)

                _drain_extract(g0 + 1, 1, sem1)
                return 0

            lax.fori_loop(0, _NG // 2, _pair_body, 0)

            pltpu.sync_copy(out_v, o.at[pl.ds(cbase, _CHUNK)])


def kernel(feat0, feat1, feat2, feat3,
           table_feat0, table_feat1, table_feat2, table_feat3):
    return _gather4(
        table_feat0.T, table_feat1.T, table_feat2.T, table_feat3.T,
        feat0.astype(jnp.int32), feat1.astype(jnp.int32),
        feat2.astype(jnp.int32), feat3.astype(jnp.int32),
    )


# speedup vs baseline: 3.5079x; 1.0191x over previous
import functools

import jax
import jax.numpy as jnp
from jax import lax
from jax.experimental import pallas as pl
from jax.experimental.pallas import tpu as pltpu
from jax.experimental.pallas import tpu_sc as plsc

BATCH = 16384
VOCAB = 1000000
DIM = 16
NFIELDS = 4

_info = plsc.get_sparse_core_info()
_NC = _info.num_cores
_NS = _info.num_subcores
_NW = _NC * _NS
_BPW = BATCH // _NW
_CHUNK = 256
_NG = _CHUNK // 16

_mesh = plsc.VectorSubcoreMesh(core_axis_name="c", subcore_axis_name="s")


@functools.partial(
    pl.kernel,
    mesh=_mesh,
    out_type=tuple(
        jax.ShapeDtypeStruct((BATCH, DIM), jnp.float32) for _ in range(NFIELDS)
    ),
    scratch_types=[
        pltpu.VMEM((_CHUNK,), jnp.int32),
        pltpu.VMEM((2 * 16 * DIM, 128), jnp.float32),
        pltpu.VMEM((_CHUNK, DIM), jnp.float32),
        pltpu.SemaphoreType.DMA,
        pltpu.SemaphoreType.DMA,
        pltpu.SemaphoreType.DMA,
    ],
    compiler_params=pltpu.CompilerParams(needs_layout_passes=False),
)
def _gather4(t0, t1, t2, t3, i0, i1, i2, i3, o0, o1, o2, o3,
             idx_v, win_v, out_v, sem0, sem1, osem):
    wid = lax.axis_index("s") * _NC + lax.axis_index("c")
    base = wid * _BPW
    lane = lax.iota(jnp.int32, 16)

    for fi, (t, i, o) in enumerate(
        ((t0, i0, o0), (t1, i1, o1), (t2, i2, o2), (t3, i3, o3))):

        def _fire(g, par, sem):
            kv = idx_v[pl.ds(g * 16, 16)]
            tc128 = jax.lax.shift_right_logical(kv, 7) * 128
            for j in range(16):
                start = pl.multiple_of(tc128[j], 128)
                pltpu.async_copy(
                    t.at[:, pl.ds(start, 128)],
                    win_v.at[pl.ds((par * 16 + j) * DIM, DIM), :],
                    sem,
                )

        def _drain_extract(g, par, sem):
            for j in range(16):
                pltpu.make_async_copy(
                    t.at[:, pl.ds(0, 128)],
                    win_v.at[pl.ds((par * 16 + j) * DIM, DIM), :],
                    sem,
                ).wait()
            kv = idx_v[pl.ds(g * 16, 16)]
            lv = kv & 127
            for j in range(16):
                vals = plsc.load_gather(
                    win_v,
                    [(par * 16 + j) * DIM + lane,
                     jnp.full((16,), lv[j], jnp.int32)],
                )
                out_v[g * 16 + j] = vals

        for c in range(_BPW // _CHUNK):
            cbase = base + c * _CHUNK
            pltpu.sync_copy(i.at[pl.ds(cbase, _CHUNK)], idx_v)
            if fi > 0 or c > 0:
                pltpu.make_async_copy(
                    out_v, o.at[pl.ds(cbase, _CHUNK)], osem).wait()

            _fire(0, 0, sem0)

            def _pair_body(h, _):
                g0 = h * 2
                _fire(g0 + 1, 1, sem1)
                _drain_extract(g0, 0, sem0)

                @pl.when(h + 1 < _NG // 2)
                def _():
                    _fire(g0 + 2, 0, sem0)

                _drain_extract(g0 + 1, 1, sem1)
                return 0

            lax.fori_loop(0, _NG // 2, _pair_body, 0)

            if fi < NFIELDS - 1 or c < _BPW // _CHUNK - 1:
                pltpu.async_copy(out_v, o.at[pl.ds(cbase, _CHUNK)], osem)
            else:
                pltpu.sync_copy(out_v, o.at[pl.ds(cbase, _CHUNK)])


def kernel(feat0, feat1, feat2, feat3,
           table_feat0, table_feat1, table_feat2, table_feat3):
    return _gather4(
        table_feat0.T, table_feat1.T, table_feat2.T, table_feat3.T,
        feat0.astype(jnp.int32), feat1.astype(jnp.int32),
        feat2.astype(jnp.int32), feat3.astype(jnp.int32),
    )


# speedup vs baseline: 3.7995x; 1.0831x over previous
import functools

import jax
import jax.numpy as jnp
from jax import lax
from jax.experimental import pallas as pl
from jax.experimental.pallas import tpu as pltpu
from jax.experimental.pallas import tpu_sc as plsc

BATCH = 16384
VOCAB = 1000000
DIM = 16
NFIELDS = 4

_info = plsc.get_sparse_core_info()
_NC = _info.num_cores
_NS = _info.num_subcores
_NW = _NC * _NS
_BPW = BATCH // _NW
_CHUNK = 256
_NG = _CHUNK // 16

_mesh = plsc.VectorSubcoreMesh(core_axis_name="c", subcore_axis_name="s")


@functools.partial(
    pl.kernel,
    mesh=_mesh,
    out_type=tuple(
        jax.ShapeDtypeStruct((DIM, BATCH), jnp.float32) for _ in range(NFIELDS)
    ),
    scratch_types=[
        pltpu.VMEM((_CHUNK,), jnp.int32),
        pltpu.VMEM((2 * 16 * DIM, 128), jnp.float32),
        pltpu.VMEM((DIM, _CHUNK), jnp.float32),
        pltpu.SemaphoreType.DMA,
        pltpu.SemaphoreType.DMA,
        pltpu.SemaphoreType.DMA,
    ],
    compiler_params=pltpu.CompilerParams(needs_layout_passes=False),
)
def _gather4(t0, t1, t2, t3, i0, i1, i2, i3, o0, o1, o2, o3,
             idx_v, win_v, out_v, sem0, sem1, osem):
    wid = lax.axis_index("s") * _NC + lax.axis_index("c")
    base = wid * _BPW
    lane = lax.iota(jnp.int32, 16)

    for fi, (t, i, o) in enumerate(
        ((t0, i0, o0), (t1, i1, o1), (t2, i2, o2), (t3, i3, o3))):

        def _fire(g, par, sem):
            kv = idx_v[pl.ds(g * 16, 16)]
            tc128 = jax.lax.shift_right_logical(kv, 7) * 128
            for j in range(16):
                start = pl.multiple_of(tc128[j], 128)
                pltpu.async_copy(
                    t.at[:, pl.ds(start, 128)],
                    win_v.at[pl.ds((par * 16 + j) * DIM, DIM), :],
                    sem,
                )

        def _drain_extract(g, par, sem):
            for j in range(16):
                pltpu.make_async_copy(
                    t.at[:, pl.ds(0, 128)],
                    win_v.at[pl.ds((par * 16 + j) * DIM, DIM), :],
                    sem,
                ).wait()
            kv = idx_v[pl.ds(g * 16, 16)]
            lv = kv & 127
            for j in range(16):
                vals = plsc.load_gather(
                    win_v,
                    [(par * 16 + j) * DIM + lane,
                     jnp.full((16,), lv[j], jnp.int32)],
                )
                plsc.store_scatter(
                    out_v, [lane, jnp.full((16,), g * 16 + j, jnp.int32)],
                    vals)

        for c in range(_BPW // _CHUNK):
            cbase = base + c * _CHUNK
            pltpu.sync_copy(i.at[pl.ds(cbase, _CHUNK)], idx_v)
            if fi > 0 or c > 0:
                pltpu.make_async_copy(
                    out_v, o.at[:, pl.ds(cbase, _CHUNK)], osem).wait()

            _fire(0, 0, sem0)

            def _pair_body(h, _):
                g0 = h * 2
                _fire(g0 + 1, 1, sem1)
                _drain_extract(g0, 0, sem0)

                @pl.when(h + 1 < _NG // 2)
                def _():
                    _fire(g0 + 2, 0, sem0)

                _drain_extract(g0 + 1, 1, sem1)
                return 0

            lax.fori_loop(0, _NG // 2, _pair_body, 0)

            if fi < NFIELDS - 1 or c < _BPW // _CHUNK - 1:
                pltpu.async_copy(out_v, o.at[:, pl.ds(cbase, _CHUNK)], osem)
            else:
                pltpu.sync_copy(out_v, o.at[:, pl.ds(cbase, _CHUNK)])


def kernel(feat0, feat1, feat2, feat3,
           table_feat0, table_feat1, table_feat2, table_feat3):
    outs = _gather4(
        table_feat0.T, table_feat1.T, table_feat2.T, table_feat3.T,
        feat0.astype(jnp.int32), feat1.astype(jnp.int32),
        feat2.astype(jnp.int32), feat3.astype(jnp.int32),
    )
    return tuple(o.T for o in outs)
